# baseline (speedup 1.0000x reference)
import functools

import jax
import jax.numpy as jnp
from jax.experimental import pallas as pl
from jax.experimental.pallas import tpu as pltpu

B, D, K, C = 32, 1024, 8, 1000
DB = D // K
MAX_ROUNDS, MIN_ROUNDS, TOPK, IG_STEPS = 4, 1, 4, 8
ENT_T, ENT_H, CONF_T = 0.65, 0.05, 0.7
ALPHA, BETA = 1.0, 1.0
SNR_DB, PT, EPS = 10.0, 1.0, 1e-8
GAMMA = 10.0 ** (SNR_DB / 10.0)
SB = B * IG_STEPS


def _fwd(z, W1, b1, W2, b2, Whp, bhp):
    a = jnp.dot(z, W1, preferred_element_type=jnp.float32) + b1
    h1 = jnp.maximum(a, 0.0)
    h2 = jnp.dot(h1, W2, preferred_element_type=jnp.float32) + b2
    logits = jnp.dot(h2, Whp, preferred_element_type=jnp.float32) + bhp
    return logits


def _stats(logits):
    m = jnp.max(logits, axis=1, keepdims=True)
    ex = jnp.exp(logits - m)
    zs = jnp.sum(ex, axis=1, keepdims=True)
    p = ex / zs
    ent = -jnp.sum(p * jnp.log(jnp.maximum(p, 1e-8)), axis=1, keepdims=True)
    conf = jnp.max(p, axis=1, keepdims=True)
    lane = jax.lax.broadcasted_iota(jnp.int32, (B, C), 1)
    pred = jnp.min(jnp.where(p == conf, lane, C), axis=1, keepdims=True)
    return ent, conf, pred


def _blocksum(v):
    return jnp.concatenate(
        [jnp.sum(v[:, k * DB:(k + 1) * DB], axis=1, keepdims=True)
         for k in range(K)], axis=1)


def _blockbcast(v):
    return jnp.concatenate(
        [jnp.broadcast_to(v[:, k:k + 1], (B, DB)) for k in range(K)], axis=1)


def _harq_body(z0_ref, y_ref, obs_ref, W1_hbm, b1_ref, W2_hbm, b2_ref,
               Wh_hbm, bh_ref, g_ref,
               z_out, logits_out, stats_out,
               w1_ref, w2_ref, wh_ref, sems):
    c1 = pltpu.make_async_copy(W1_hbm, w1_ref, sems.at[0])
    c2 = pltpu.make_async_copy(W2_hbm, w2_ref, sems.at[1])
    c3 = pltpu.make_async_copy(Wh_hbm, wh_ref, sems.at[2])
    c1.start()
    c2.start()
    c3.start()

    b1 = b1_ref[...]
    b2 = b2_ref[...]
    bhp = bh_ref[...]
    g = g_ref[...]

    z = z0_ref[...]
    c1.wait()
    W1 = w1_ref[...]
    a1 = jnp.dot(z, W1, preferred_element_type=jnp.float32) + b1
    h1 = jnp.maximum(a1, 0.0)
    c2.wait()
    W2 = w2_ref[...]
    h2 = jnp.dot(h1, W2, preferred_element_type=jnp.float32) + b2
    c3.wait()
    Whp = wh_ref[...]
    logits = jnp.dot(h2, Whp, preferred_element_type=jnp.float32) + bhp
    ent, conf, pred = _stats(logits)
    rounds_used = jnp.ones((B, 1), jnp.float32)
    block_rounds = jnp.ones((B, K), jnp.float32)
    blocks_retx = jnp.zeros((B, 1), jnp.float32)
    done = jnp.logical_and(ent <= ENT_T - ENT_H, conf >= CONF_T)
    done = done.astype(jnp.float32)

    kidx = jax.lax.broadcasted_iota(jnp.int32, (B, K), 1)
    lane = jax.lax.broadcasted_iota(jnp.int32, (B, C), 1)

    for r in range(2, MAX_ROUNDS + 1):
        y = y_ref[r - 2]
        delta = y - z

        oh = (lane == pred).astype(jnp.float32)
        vb = jax.lax.dot_general(oh, Whp, (((1,), (1,)), ((), ())),
                                 preferred_element_type=jnp.float32)
        u = jax.lax.dot_general(vb, W2, (((1,), (1,)), ((), ())),
                                preferred_element_type=jnp.float32)

        zs = jnp.concatenate(
            [z + (float(s) / IG_STEPS) * delta
             for s in range(1, IG_STEPS + 1)], axis=0)
        a = jnp.dot(zs, W1, preferred_element_type=jnp.float32) + b1
        p_in = jnp.concatenate([u] * IG_STEPS, axis=0)
        p_in = jnp.where(a > 0.0, p_in, 0.0)
        dz = jax.lax.dot_general(p_in, W1, (((1,), (1,)), ((), ())),
                                 preferred_element_type=jnp.float32)
        total = dz[0:B]
        for s in range(1, IG_STEPS):
            total = total + dz[s * B:(s + 1) * B]
        avg = total / IG_STEPS
        contrib = jnp.abs(_blocksum(delta * avg))

        snr_blk_db = 10.0 * jnp.log10(jnp.maximum(block_rounds * GAMMA, 1e-8))
        c = contrib / (jnp.max(contrib, axis=1, keepdims=True) + EPS)
        smin = jnp.min(snr_blk_db, axis=1, keepdims=True)
        smax = jnp.max(snr_blk_db, axis=1, keepdims=True)
        sn = (snr_blk_db - smin) / (smax - smin + EPS)
        score = ALPHA * c - BETA * sn

        rank = jnp.zeros((B, K), jnp.float32)
        for j in range(K):
            sj = score[:, j:j + 1]
            gt = (sj > score).astype(jnp.float32)
            eq = jnp.logical_and(sj == score, j < kidx).astype(jnp.float32)
            rank = rank + gt + eq
        mask = (rank < float(TOPK)).astype(jnp.float32)

        active = 1.0 - done
        updm = mask * active
        upd = _blockbcast(updm)

        new_obs = obs_ref[r - 2]
        combined = g * z + (1.0 - g) * new_obs
        z = z * (1.0 - upd) + combined * upd
        block_rounds = block_rounds + updm
        blocks_retx = blocks_retx + active * float(TOPK)
        rounds_used = jnp.where(done > 0.0, rounds_used, jnp.float32(r))

        logits = _fwd(z, W1, b1, W2, b2, Whp, bhp)
        ent, conf, pred = _stats(logits)
        stop = jnp.logical_and(
            jnp.logical_and(rounds_used >= MIN_ROUNDS, ent <= ENT_T - ENT_H),
            conf >= CONF_T).astype(jnp.float32)
        done = jnp.maximum(done, stop)

    z_out[...] = z
    logits_out[...] = logits
    stats_out[...] = jnp.concatenate(
        [ent, conf, rounds_used, blocks_retx], axis=1)


_NOISE_CACHE = None


def _noise_consts():
    global _NOISE_CACHE
    if _NOISE_CACHE is None:
        with jax.ensure_compile_time_eval():
            nkey = jax.random.key(42)
            nscale = jnp.sqrt(PT / GAMMA)

            def noise(j, shape):
                return jax.random.normal(jax.random.fold_in(nkey, j), shape,
                                         dtype=jnp.float32) * nscale

            nx = jnp.stack([noise(0, (B, D))]
                           + [noise(10 + r, (B, D))
                              for r in range(2, MAX_ROUNDS + 1)])
            nb = jnp.stack([noise(100 + r, (B * K, DB)).reshape(B, D)
                            for r in range(2, MAX_ROUNDS + 1)])
        _NOISE_CACHE = (jax.block_until_ready(nx), jax.block_until_ready(nb))
    return _NOISE_CACHE


@functools.partial(jax.jit, static_argnames=())
def kernel(x_tx, xb_tx, W1, b1, W2, b2, Wh, bh, gate):
    nx, nb = _noise_consts()
    power = jnp.mean(x_tx * x_tx, axis=-1, keepdims=True)
    xn = x_tx * jnp.sqrt(PT / (power + EPS))
    xb2 = xb_tx.reshape(B * K, DB)
    pb = jnp.mean(xb2 * xb2, axis=-1, keepdims=True)
    xnb = (xb2 * jnp.sqrt(PT / (pb + EPS))).reshape(B, D)

    z0 = xn + nx[0]
    y = xn[None] + nx[1:]
    obs = xnb[None] + nb
    g_t = jax.nn.sigmoid(jnp.tile(gate, K)).reshape(1, D)

    hbm = pl.BlockSpec(memory_space=pltpu.MemorySpace.HBM)
    vmem = pl.BlockSpec(memory_space=pltpu.MemorySpace.VMEM)
    z, logits, stats = pl.pallas_call(
        _harq_body,
        in_specs=[vmem, vmem, vmem, hbm, vmem, hbm, vmem, hbm, vmem, vmem],
        out_shape=(
            jax.ShapeDtypeStruct((B, D), jnp.float32),
            jax.ShapeDtypeStruct((B, C), jnp.float32),
            jax.ShapeDtypeStruct((B, 4), jnp.float32),
        ),
        scratch_shapes=[
            pltpu.VMEM((D, D), jnp.float32),
            pltpu.VMEM((D, D), jnp.float32),
            pltpu.VMEM((D, C), jnp.float32),
            pltpu.SemaphoreType.DMA((3,)),
        ],
    )(z0, y, obs, W1, b1.reshape(1, D), W2, b2.reshape(1, D),
      Wh, bh.reshape(1, C), g_t)

    ent = stats[:, 0]
    conf = stats[:, 1]
    rounds_used = stats[:, 2]
    blocks_retx = stats[:, 3]
    return z, logits, ent, conf, rounds_used, blocks_retx


# speedup vs baseline: 1.0018x; 1.0018x over previous
import functools

import jax
import jax.numpy as jnp
from jax.experimental import pallas as pl
from jax.experimental.pallas import tpu as pltpu

B, D, K, C = 32, 1024, 8, 1000
DB = D // K
MAX_ROUNDS, MIN_ROUNDS, TOPK, IG_STEPS = 4, 1, 4, 8
ENT_T, ENT_H, CONF_T = 0.65, 0.05, 0.7
ALPHA, BETA = 1.0, 1.0
SNR_DB, PT, EPS = 10.0, 1.0, 1e-8
GAMMA = 10.0 ** (SNR_DB / 10.0)
SB = B * IG_STEPS


def _fwd(z, W1, b1, W2, b2, Whp, bhp):
    a = jnp.dot(z, W1, preferred_element_type=jnp.float32) + b1
    h1 = jnp.maximum(a, 0.0)
    h2 = jnp.dot(h1, W2, preferred_element_type=jnp.float32) + b2
    logits = jnp.dot(h2, Whp, preferred_element_type=jnp.float32) + bhp
    return logits


def _stats(logits):
    m = jnp.max(logits, axis=1, keepdims=True)
    ex = jnp.exp(logits - m)
    zs = jnp.sum(ex, axis=1, keepdims=True)
    p = ex / zs
    ent = -jnp.sum(p * jnp.log(jnp.maximum(p, 1e-8)), axis=1, keepdims=True)
    conf = jnp.max(p, axis=1, keepdims=True)
    lane = jax.lax.broadcasted_iota(jnp.int32, (B, C), 1)
    pred = jnp.min(jnp.where(p == conf, lane, C), axis=1, keepdims=True)
    return ent, conf, pred


def _blocksum(v):
    return jnp.concatenate(
        [jnp.sum(v[:, k * DB:(k + 1) * DB], axis=1, keepdims=True)
         for k in range(K)], axis=1)


def _blockbcast(v):
    return jnp.concatenate(
        [jnp.broadcast_to(v[:, k:k + 1], (B, DB)) for k in range(K)], axis=1)


def _harq_body(z0_ref, y_ref, obs_ref, W1_hbm, b1_ref, W2_hbm, b2_ref,
               Wh_hbm, bh_ref, g_ref,
               z_out, logits_out, stats_out,
               w1_ref, w2_ref, wh_ref, sems):
    c1 = pltpu.make_async_copy(W1_hbm, w1_ref, sems.at[0])
    c2 = pltpu.make_async_copy(W2_hbm, w2_ref, sems.at[1])
    c3 = pltpu.make_async_copy(Wh_hbm, wh_ref, sems.at[2])
    c1.start()
    c2.start()
    c3.start()

    b1 = b1_ref[...]
    b2 = b2_ref[...]
    bhp = bh_ref[...]
    g = g_ref[...]

    z = z0_ref[...]
    c1.wait()
    W1 = w1_ref[...]
    a1 = jnp.dot(z, W1, preferred_element_type=jnp.float32) + b1
    h1 = jnp.maximum(a1, 0.0)
    c2.wait()
    W2 = w2_ref[...]
    h2 = jnp.dot(h1, W2, preferred_element_type=jnp.float32) + b2
    c3.wait()
    Whp = wh_ref[...]
    logits = jnp.dot(h2, Whp, preferred_element_type=jnp.float32) + bhp
    ent, conf, pred = _stats(logits)
    rounds_used = jnp.ones((B, 1), jnp.float32)
    block_rounds = jnp.ones((B, K), jnp.float32)
    blocks_retx = jnp.zeros((B, 1), jnp.float32)
    done = jnp.logical_and(ent <= ENT_T - ENT_H, conf >= CONF_T)
    done = done.astype(jnp.float32)

    kidx = jax.lax.broadcasted_iota(jnp.int32, (B, K), 1)
    lane = jax.lax.broadcasted_iota(jnp.int32, (B, C), 1)

    for r in range(2, MAX_ROUNDS + 1):
        y = y_ref[r - 2]
        delta = y - z

        oh = (lane == pred).astype(jnp.float32)
        vb = jax.lax.dot_general(oh, Whp, (((1,), (1,)), ((), ())),
                                 preferred_element_type=jnp.float32)
        u = jax.lax.dot_general(vb, W2, (((1,), (1,)), ((), ())),
                                preferred_element_type=jnp.float32)

        zs = jnp.concatenate(
            [z + (float(s) / IG_STEPS) * delta
             for s in range(1, IG_STEPS + 1)], axis=0)
        a = jnp.dot(zs, W1, preferred_element_type=jnp.float32) + b1
        p_in = jnp.concatenate([u] * IG_STEPS, axis=0)
        p_in = jnp.where(a > 0.0, p_in, 0.0)
        dz = jax.lax.dot_general(p_in, W1, (((1,), (1,)), ((), ())),
                                 preferred_element_type=jnp.float32)
        total = dz[0:B]
        for s in range(1, IG_STEPS):
            total = total + dz[s * B:(s + 1) * B]
        avg = total / IG_STEPS
        contrib = jnp.abs(_blocksum(delta * avg))

        snr_blk_db = 10.0 * jnp.log10(jnp.maximum(block_rounds * GAMMA, 1e-8))
        c = contrib / (jnp.max(contrib, axis=1, keepdims=True) + EPS)
        smin = jnp.min(snr_blk_db, axis=1, keepdims=True)
        smax = jnp.max(snr_blk_db, axis=1, keepdims=True)
        sn = (snr_blk_db - smin) / (smax - smin + EPS)
        score = ALPHA * c - BETA * sn

        rank = jnp.zeros((B, K), jnp.float32)
        for j in range(K):
            sj = score[:, j:j + 1]
            gt = (sj > score).astype(jnp.float32)
            eq = jnp.logical_and(sj == score, j < kidx).astype(jnp.float32)
            rank = rank + gt + eq
        mask = (rank < float(TOPK)).astype(jnp.float32)

        active = 1.0 - done
        updm = mask * active
        upd = _blockbcast(updm)

        new_obs = obs_ref[r - 2]
        combined = g * z + (1.0 - g) * new_obs
        z = z * (1.0 - upd) + combined * upd
        block_rounds = block_rounds + updm
        blocks_retx = blocks_retx + active * float(TOPK)
        rounds_used = jnp.where(done > 0.0, rounds_used, jnp.float32(r))

        logits = _fwd(z, W1, b1, W2, b2, Whp, bhp)
        ent, conf, pred = _stats(logits)
        stop = jnp.logical_and(
            jnp.logical_and(rounds_used >= MIN_ROUNDS, ent <= ENT_T - ENT_H),
            conf >= CONF_T).astype(jnp.float32)
        done = jnp.maximum(done, stop)

    z_out[...] = z
    logits_out[...] = logits
    stats_out[...] = jnp.concatenate(
        [ent, conf, rounds_used, blocks_retx], axis=1)


_NOISE_CACHE = None


def _noise_consts():
    global _NOISE_CACHE
    if _NOISE_CACHE is not None:
        return _NOISE_CACHE

    def build():
        nkey = jax.random.key(42)
        nscale = jnp.sqrt(PT / GAMMA)

        def noise(j, shape):
            return jax.random.normal(jax.random.fold_in(nkey, j), shape,
                                     dtype=jnp.float32) * nscale

        nx = jnp.stack([noise(0, (B, D))]
                       + [noise(10 + r, (B, D))
                          for r in range(2, MAX_ROUNDS + 1)])
        nb = jnp.stack([noise(100 + r, (B * K, DB)).reshape(B, D)
                        for r in range(2, MAX_ROUNDS + 1)])
        return nx, nb

    try:
        with jax.ensure_compile_time_eval():
            nx, nb = build()
        _NOISE_CACHE = (jax.block_until_ready(nx), jax.block_until_ready(nb))
        return _NOISE_CACHE
    except Exception:
        return build()


@functools.partial(jax.jit, static_argnames=())
def kernel(x_tx, xb_tx, W1, b1, W2, b2, Wh, bh, gate):
    nx, nb = _noise_consts()
    power = jnp.mean(x_tx * x_tx, axis=-1, keepdims=True)
    xn = x_tx * jnp.sqrt(PT / (power + EPS))
    xb2 = xb_tx.reshape(B * K, DB)
    pb = jnp.mean(xb2 * xb2, axis=-1, keepdims=True)
    xnb = (xb2 * jnp.sqrt(PT / (pb + EPS))).reshape(B, D)

    z0 = xn + nx[0]
    y = xn[None] + nx[1:]
    obs = xnb[None] + nb
    g_t = jax.nn.sigmoid(jnp.tile(gate, K)).reshape(1, D)

    hbm = pl.BlockSpec(memory_space=pltpu.MemorySpace.HBM)
    vmem = pl.BlockSpec(memory_space=pltpu.MemorySpace.VMEM)
    z, logits, stats = pl.pallas_call(
        _harq_body,
        in_specs=[vmem, vmem, vmem, hbm, vmem, hbm, vmem, hbm, vmem, vmem],
        out_shape=(
            jax.ShapeDtypeStruct((B, D), jnp.float32),
            jax.ShapeDtypeStruct((B, C), jnp.float32),
            jax.ShapeDtypeStruct((B, 4), jnp.float32),
        ),
        scratch_shapes=[
            pltpu.VMEM((D, D), jnp.float32),
            pltpu.VMEM((D, D), jnp.float32),
            pltpu.VMEM((D, C), jnp.float32),
            pltpu.SemaphoreType.DMA((3,)),
        ],
    )(z0, y, obs, W1, b1.reshape(1, D), W2, b2.reshape(1, D),
      Wh, bh.reshape(1, C), g_t)

    ent = stats[:, 0]
    conf = stats[:, 1]
    rounds_used = stats[:, 2]
    blocks_retx = stats[:, 3]
    return z, logits, ent, conf, rounds_used, blocks_retx


# speedup vs baseline: 1.1369x; 1.1349x over previous
import functools

import jax
import jax.numpy as jnp
from jax.experimental import pallas as pl
from jax.experimental.pallas import tpu as pltpu

B, D, K, C = 32, 1024, 8, 1000
DB = D // K
MAX_ROUNDS, MIN_ROUNDS, TOPK, IG_STEPS = 4, 1, 4, 8
ENT_T, ENT_H, CONF_T = 0.65, 0.05, 0.7
ALPHA, BETA = 1.0, 1.0
SNR_DB, PT, EPS = 10.0, 1.0, 1e-8
GAMMA = 10.0 ** (SNR_DB / 10.0)
SB = B * IG_STEPS


def _fwd(z, W1, b1, W2, b2, Whp, bhp):
    a = jnp.dot(z, W1, preferred_element_type=jnp.float32) + b1
    h1 = jnp.maximum(a, 0.0)
    h2 = jnp.dot(h1, W2, preferred_element_type=jnp.float32) + b2
    logits = jnp.dot(h2, Whp, preferred_element_type=jnp.float32) + bhp
    return logits


def _stats(logits):
    m = jnp.max(logits, axis=1, keepdims=True)
    ex = jnp.exp(logits - m)
    zs = jnp.sum(ex, axis=1, keepdims=True)
    p = ex / zs
    ent = -jnp.sum(p * jnp.log(jnp.maximum(p, 1e-8)), axis=1, keepdims=True)
    conf = jnp.max(p, axis=1, keepdims=True)
    lane = jax.lax.broadcasted_iota(jnp.int32, (B, C), 1)
    pred = jnp.min(jnp.where(p == conf, lane, C), axis=1, keepdims=True)
    return ent, conf, pred


def _blocksum(v):
    return jnp.concatenate(
        [jnp.sum(v[:, k * DB:(k + 1) * DB], axis=1, keepdims=True)
         for k in range(K)], axis=1)


def _blockbcast(v):
    return jnp.concatenate(
        [jnp.broadcast_to(v[:, k:k + 1], (B, DB)) for k in range(K)], axis=1)


def _harq_body(xn_ref, xnb_ref, nx_ref, nb_ref, W1_hbm, b1_ref, W2_hbm,
               b2_ref, Wh_hbm, bh_ref, gate_ref,
               z_out, logits_out, ent_out, conf_out, ru_out, retx_out,
               w1_ref, w2_ref, wh_ref, sems):
    c1 = pltpu.make_async_copy(W1_hbm, w1_ref, sems.at[0])
    c2 = pltpu.make_async_copy(W2_hbm, w2_ref, sems.at[1])
    c3 = pltpu.make_async_copy(Wh_hbm, wh_ref, sems.at[2])
    c1.start()
    c2.start()
    c3.start()

    b1 = b1_ref[...]
    b2 = b2_ref[...]
    bhp = bh_ref[...]
    gs = jax.nn.sigmoid(gate_ref[...])
    g = jnp.concatenate([gs] * K, axis=1)

    xn = xn_ref[...]
    xnb = xnb_ref[...]

    z = xn + nx_ref[0]
    c1.wait()
    W1 = w1_ref[...]
    a1 = jnp.dot(z, W1, preferred_element_type=jnp.float32) + b1
    h1 = jnp.maximum(a1, 0.0)
    c2.wait()
    W2 = w2_ref[...]
    h2 = jnp.dot(h1, W2, preferred_element_type=jnp.float32) + b2
    c3.wait()
    Whp = wh_ref[...]
    logits = jnp.dot(h2, Whp, preferred_element_type=jnp.float32) + bhp
    ent, conf, pred = _stats(logits)
    rounds_used = jnp.ones((B, 1), jnp.float32)
    block_rounds = jnp.ones((B, K), jnp.float32)
    blocks_retx = jnp.zeros((B, 1), jnp.float32)
    done = jnp.logical_and(ent <= ENT_T - ENT_H, conf >= CONF_T)
    done = done.astype(jnp.float32)

    kidx = jax.lax.broadcasted_iota(jnp.int32, (B, K), 1)
    lane = jax.lax.broadcasted_iota(jnp.int32, (B, C), 1)

    for r in range(2, MAX_ROUNDS + 1):
        y = xn + nx_ref[r - 1]
        delta = y - z

        oh = (lane == pred).astype(jnp.float32)
        vb = jax.lax.dot_general(oh, Whp, (((1,), (1,)), ((), ())),
                                 preferred_element_type=jnp.float32)
        u = jax.lax.dot_general(vb, W2, (((1,), (1,)), ((), ())),
                                preferred_element_type=jnp.float32)

        zs = jnp.concatenate(
            [z + (float(s) / IG_STEPS) * delta
             for s in range(1, IG_STEPS + 1)], axis=0)
        a = jnp.dot(zs, W1, preferred_element_type=jnp.float32) + b1
        p_in = jnp.concatenate([u] * IG_STEPS, axis=0)
        p_in = jnp.where(a > 0.0, p_in, 0.0)
        dz = jax.lax.dot_general(p_in, W1, (((1,), (1,)), ((), ())),
                                 preferred_element_type=jnp.float32)
        total = dz[0:B]
        for s in range(1, IG_STEPS):
            total = total + dz[s * B:(s + 1) * B]
        avg = total / IG_STEPS
        contrib = jnp.abs(_blocksum(delta * avg))

        snr_blk_db = 10.0 * jnp.log10(jnp.maximum(block_rounds * GAMMA, 1e-8))
        c = contrib / (jnp.max(contrib, axis=1, keepdims=True) + EPS)
        smin = jnp.min(snr_blk_db, axis=1, keepdims=True)
        smax = jnp.max(snr_blk_db, axis=1, keepdims=True)
        sn = (snr_blk_db - smin) / (smax - smin + EPS)
        score = ALPHA * c - BETA * sn

        rank = jnp.zeros((B, K), jnp.float32)
        for j in range(K):
            sj = score[:, j:j + 1]
            gt = (sj > score).astype(jnp.float32)
            eq = jnp.logical_and(sj == score, j < kidx).astype(jnp.float32)
            rank = rank + gt + eq
        mask = (rank < float(TOPK)).astype(jnp.float32)

        active = 1.0 - done
        updm = mask * active
        upd = _blockbcast(updm)

        new_obs = xnb + nb_ref[r - 2]
        combined = g * z + (1.0 - g) * new_obs
        z = z * (1.0 - upd) + combined * upd
        block_rounds = block_rounds + updm
        blocks_retx = blocks_retx + active * float(TOPK)
        rounds_used = jnp.where(done > 0.0, rounds_used, jnp.float32(r))

        logits = _fwd(z, W1, b1, W2, b2, Whp, bhp)
        ent, conf, pred = _stats(logits)
        stop = jnp.logical_and(
            jnp.logical_and(rounds_used >= MIN_ROUNDS, ent <= ENT_T - ENT_H),
            conf >= CONF_T).astype(jnp.float32)
        done = jnp.maximum(done, stop)

    z_out[...] = z
    logits_out[...] = logits
    ent_out[...] = ent[:, 0]
    conf_out[...] = conf[:, 0]
    ru_out[...] = rounds_used[:, 0]
    retx_out[...] = blocks_retx[:, 0]


_NOISE_CACHE = None


def _noise_consts():
    global _NOISE_CACHE
    if _NOISE_CACHE is not None:
        return _NOISE_CACHE

    def build():
        nkey = jax.random.key(42)
        nscale = jnp.sqrt(PT / GAMMA)

        def noise(j, shape):
            return jax.random.normal(jax.random.fold_in(nkey, j), shape,
                                     dtype=jnp.float32) * nscale

        nx = jnp.stack([noise(0, (B, D))]
                       + [noise(10 + r, (B, D))
                          for r in range(2, MAX_ROUNDS + 1)])
        nb = jnp.stack([noise(100 + r, (B * K, DB)).reshape(B, D)
                        for r in range(2, MAX_ROUNDS + 1)])
        return nx, nb

    try:
        with jax.ensure_compile_time_eval():
            nx, nb = build()
        _NOISE_CACHE = (jax.block_until_ready(nx), jax.block_until_ready(nb))
        return _NOISE_CACHE
    except Exception:
        return build()


@functools.partial(jax.jit, static_argnames=())
def kernel(x_tx, xb_tx, W1, b1, W2, b2, Wh, bh, gate):
    nx, nb = _noise_consts()
    power = jnp.mean(x_tx * x_tx, axis=-1, keepdims=True)
    xn = x_tx * jnp.sqrt(PT / (power + EPS))
    xb2 = xb_tx.reshape(B * K, DB)
    pb = jnp.mean(xb2 * xb2, axis=-1, keepdims=True)
    xnb = (xb2 * jnp.sqrt(PT / (pb + EPS))).reshape(B, D)

    hbm = pl.BlockSpec(memory_space=pltpu.MemorySpace.HBM)
    vmem = pl.BlockSpec(memory_space=pltpu.MemorySpace.VMEM)
    z, logits, ent, conf, rounds_used, blocks_retx = pl.pallas_call(
        _harq_body,
        in_specs=[vmem, vmem, vmem, vmem, hbm, vmem, hbm, vmem, hbm,
                  vmem, vmem],
        out_shape=(
            jax.ShapeDtypeStruct((B, D), jnp.float32),
            jax.ShapeDtypeStruct((B, C), jnp.float32),
            jax.ShapeDtypeStruct((B,), jnp.float32),
            jax.ShapeDtypeStruct((B,), jnp.float32),
            jax.ShapeDtypeStruct((B,), jnp.float32),
            jax.ShapeDtypeStruct((B,), jnp.float32),
        ),
        scratch_shapes=[
            pltpu.VMEM((D, D), jnp.float32),
            pltpu.VMEM((D, D), jnp.float32),
            pltpu.VMEM((D, C), jnp.float32),
            pltpu.SemaphoreType.DMA((3,)),
        ],
    )(xn, xnb, nx, nb, W1, b1.reshape(1, D), W2, b2.reshape(1, D),
      Wh, bh.reshape(1, C), gate.reshape(1, DB))

    return z, logits, ent, conf, rounds_used, blocks_retx
